# baseline (speedup 1.0000x reference)
import numpy as np
import jax
import jax.numpy as jnp
from jax.experimental import pallas as pl
from jax.experimental.pallas import tpu as pltpu

N = 4096
D = 128
BLK = 512
NBLK = N // BLK
NTILES = NBLK * (NBLK + 1) // 2

_BI = np.array([bi for bi in range(NBLK) for bj in range(bi, NBLK)],
               dtype=np.int32)
_BJ = np.array([bj for bi in range(NBLK) for bj in range(bi, NBLK)],
               dtype=np.int32)


def _koleo_kernel(bi_ref, bj_ref, emb_ref, out_ref,
                  rmin_col_ref, rmin_row_ref, sqn_col_ref, sqn_row_ref):
    emb = emb_ref[...]
    sqn = jnp.sum(emb * emb, axis=1)
    sqn_col_ref[...] = sqn[:, None]
    sqn_row_ref[...] = sqn[None, :]
    rmin_col_ref[...] = jnp.full((N, 1), jnp.inf, jnp.float32)
    rmin_row_ref[...] = jnp.full((1, N), jnp.inf, jnp.float32)

    def tile(t, gmax):
        bi = bi_ref[t]
        bj = bj_ref[t]
        blk_i = emb_ref[pl.ds(bi * BLK, BLK), :]
        blk_j = emb_ref[pl.ds(bj * BLK, BLK), :]
        sqn_i = sqn_col_ref[pl.ds(bi * BLK, BLK), :]
        sqn_j = sqn_row_ref[:, pl.ds(bj * BLK, BLK)]

        dot2 = jax.lax.dot_general(
            blk_i + blk_i, blk_j, (((1,), (1,)), ((), ())),
            preferred_element_type=jnp.float32)

        sq = sqn_i + sqn_j - dot2
        masked = jnp.where(sq <= 0.0, jnp.inf, sq)

        rmin_i = jnp.min(masked, axis=1)[:, None]
        rmin_j = jnp.min(masked, axis=0)[None, :]

        isl = pl.ds(bi * BLK, BLK)
        jsl = pl.ds(bj * BLK, BLK)
        rmin_col_ref[isl, :] = jnp.minimum(rmin_col_ref[isl, :], rmin_i)
        rmin_row_ref[:, jsl] = jnp.minimum(rmin_row_ref[:, jsl], rmin_j)
        return jnp.maximum(gmax, jnp.max(sq))

    gmax = jax.lax.fori_loop(0, NTILES, tile, -jnp.inf)

    g = jnp.maximum(gmax, 0.0)
    a = rmin_col_ref[...].reshape(NBLK, BLK)
    b = rmin_row_ref[...].reshape(NBLK, BLK)
    m = jnp.minimum(jnp.minimum(a, b), g)
    d = jnp.sqrt(m)
    out_ref[...] = jnp.reshape((-1.0 / N) * jnp.sum(jnp.log(d)), (1, 1))


def kernel(embeddings):
    grid_spec = pltpu.PrefetchScalarGridSpec(
        num_scalar_prefetch=2,
        grid=(1,),
        in_specs=[pl.BlockSpec((N, D), lambda t, bi, bj: (0, 0))],
        out_specs=pl.BlockSpec((1, 1), lambda t, bi, bj: (0, 0)),
        scratch_shapes=[
            pltpu.VMEM((N, 1), jnp.float32),
            pltpu.VMEM((1, N), jnp.float32),
            pltpu.VMEM((N, 1), jnp.float32),
            pltpu.VMEM((1, N), jnp.float32),
        ],
    )
    out = pl.pallas_call(
        _koleo_kernel,
        grid_spec=grid_spec,
        out_shape=jax.ShapeDtypeStruct((1, 1), jnp.float32),
    )(jnp.asarray(_BI), jnp.asarray(_BJ), embeddings)
    return out[0, 0]
